# baseline (speedup 1.0000x reference)
import functools

import jax
import jax.numpy as jnp
from jax import lax
from jax.experimental import pallas as pl
from jax.experimental.pallas import tpu as pltpu
from jax.experimental.pallas import tpu_sc as plsc

NSTEPS = 3
NC, NS = 2, 16
NW = NC * NS
K = 96
BR = 2560


def _pad_nodes(n):
    import math
    lcm = math.lcm(NS * 8, BR)
    return -(-(n + 1) // lcm) * lcm


def _make_sc_segsum(n_pad, d, nb, gather, k=K, nb_hi=None, tc_tiling=False,
                    n_rows=None):
    rows_per_tile = n_pad // NS
    if nb_hi is None:
        nb_hi = nb
    mesh = plsc.VectorSubcoreMesh(core_axis_name="c", subcore_axis_name="s",
                                  num_cores=NC, num_subcores=NS)

    if gather is True:
        def body(vals_hbm, src_hbm, dst_hbm, zeros_hbm, out_hbm,
                 src_v, dst_v, rows_a, rows_b, acc, sg_a, sg_b, ss_a, ss_b):
            c = lax.axis_index("c")
            s = lax.axis_index("s")
            wid = c * NS + s
            pltpu.sync_copy(zeros_hbm, acc.at[pl.ds(s * rows_per_tile, rows_per_tile)])
            pltpu.sync_copy(src_hbm.at[wid], src_v)
            pltpu.sync_copy(dst_hbm.at[wid], dst_v)
            plsc.subcore_barrier()

            rows = (rows_a, rows_b)
            sg = (sg_a, sg_b)
            ss = (ss_a, ss_b)

            def pipeline(lo, hi):
                g_descs = [None] * hi
                s_descs = [None] * hi
                g_descs[lo] = pltpu.async_copy(
                    vals_hbm.at[src_v.at[lo]], rows[lo % 2], sg[lo % 2])
                for j in range(lo, hi):
                    b = j % 2
                    if j + 1 < hi:
                        if j - 1 >= lo:
                            s_descs[j - 1].wait()
                        g_descs[j + 1] = pltpu.async_copy(
                            vals_hbm.at[src_v.at[j + 1]], rows[1 - b], sg[1 - b])
                    g_descs[j].wait()
                    s_descs[j] = pltpu.async_copy(
                        rows[b], acc.at[dst_v.at[j]], ss[b], add=True)
                if hi - lo >= 2:
                    s_descs[hi - 2].wait()
                s_descs[hi - 1].wait()

            pipeline(0, nb)
            if nb_hi > nb:
                @pl.when(c == 0)
                def _():
                    pipeline(nb, nb_hi)

            plsc.subcore_barrier()
            off = c * n_pad + s * rows_per_tile
            pltpu.sync_copy(acc.at[pl.ds(s * rows_per_tile, rows_per_tile)],
                            out_hbm.at[pl.ds(off, rows_per_tile)])

        scratch = [
            pltpu.VMEM((nb_hi, k), jnp.int32),
            pltpu.VMEM((nb_hi, k), jnp.int32),
            pltpu.VMEM((k, d), jnp.float32),
            pltpu.VMEM((k, d), jnp.float32),
            pltpu.VMEM_SHARED((n_pad, d), jnp.float32),
            pltpu.SemaphoreType.DMA,
            pltpu.SemaphoreType.DMA,
            pltpu.SemaphoreType.DMA,
            pltpu.SemaphoreType.DMA,
        ]
    elif gather == "local":
        def body(cs_hbm, srcf_hbm, dst_hbm, zeros_hbm, out_hbm,
                 cs_v, srcf_v, dst_v, rows_a, rows_b, acc, ss_a, ss_b):
            c = lax.axis_index("c")
            s = lax.axis_index("s")
            wid = c * NS + s
            pltpu.sync_copy(zeros_hbm, acc.at[pl.ds(s * rows_per_tile, rows_per_tile)])
            pltpu.sync_copy(cs_hbm, cs_v)
            pltpu.sync_copy(srcf_hbm.at[wid], srcf_v)
            pltpu.sync_copy(dst_hbm.at[wid], dst_v)
            pltpu.sync_copy(zeros_hbm.at[pl.ds(0, k)], rows_a)
            pltpu.sync_copy(zeros_hbm.at[pl.ds(0, k)], rows_b)
            plsc.subcore_barrier()

            col0 = jnp.zeros((16,), jnp.int32)
            col1 = col0 + 1
            lane = lax.iota(jnp.int32, 16)

            def fill(rows_ref, j):
                for g in range(k // 16):
                    srcg = srcf_v[pl.ds(j * k + g * 16, 16)]
                    gx = plsc.load_gather(cs_v, [srcg, col0])
                    gy = plsc.load_gather(cs_v, [srcg, col1])
                    rowi = lane + (g * 16)
                    plsc.store_scatter(rows_ref, [rowi, col0], gx)
                    plsc.store_scatter(rows_ref, [rowi, col1], gy)

            rows = (rows_a, rows_b)
            ss = (ss_a, ss_b)
            s_descs = [None] * nb
            fill(rows[0], 0)
            for j in range(nb):
                b = j % 2
                s_descs[j] = pltpu.async_copy(
                    rows[b], acc.at[dst_v.at[j]], ss[b], add=True)
                if j + 1 < nb:
                    if j >= 1:
                        s_descs[j - 1].wait()
                    fill(rows[1 - b], j + 1)
            if nb >= 2:
                s_descs[nb - 2].wait()
            s_descs[nb - 1].wait()

            plsc.subcore_barrier()
            off = c * n_pad + s * rows_per_tile
            pltpu.sync_copy(acc.at[pl.ds(s * rows_per_tile, rows_per_tile)],
                            out_hbm.at[pl.ds(off, rows_per_tile)])

        scratch = [
            pltpu.VMEM((n_rows, 2), jnp.float32),
            pltpu.VMEM((nb * k,), jnp.int32),
            pltpu.VMEM((nb, k), jnp.int32),
            pltpu.VMEM((k, d), jnp.float32),
            pltpu.VMEM((k, d), jnp.float32),
            pltpu.VMEM_SHARED((n_pad, d), jnp.float32),
            pltpu.SemaphoreType.DMA,
            pltpu.SemaphoreType.DMA,
        ]
    else:
        def body(ones_hbm, dst_hbm, zeros_hbm, out_hbm,
                 dst_v, ones_v, acc):
            c = lax.axis_index("c")
            s = lax.axis_index("s")
            wid = c * NS + s
            pltpu.sync_copy(zeros_hbm, acc.at[pl.ds(s * rows_per_tile, rows_per_tile)])
            pltpu.sync_copy(dst_hbm.at[wid], dst_v)
            pltpu.sync_copy(ones_hbm, ones_v)
            plsc.subcore_barrier()

            def step(j, _):
                pltpu.sync_copy(ones_v, acc.at[dst_v.at[j]], add=True)
                return 0
            lax.fori_loop(0, nb, step, 0)

            plsc.subcore_barrier()
            off = c * n_pad + s * rows_per_tile
            pltpu.sync_copy(acc.at[pl.ds(s * rows_per_tile, rows_per_tile)],
                            out_hbm.at[pl.ds(off, rows_per_tile)])

        scratch = [
            pltpu.VMEM((nb, k), jnp.int32),
            pltpu.VMEM((k, d), jnp.float32),
            pltpu.VMEM_SHARED((n_pad, d), jnp.float32),
        ]

    return pl.kernel(
        body,
        out_type=jax.ShapeDtypeStruct((NC * n_pad, d), jnp.float32),
        mesh=mesh,
        scratch_types=scratch,
        compiler_params=pltpu.CompilerParams(use_tc_tiling_on_sc=tc_tiling),
    )


def _row_spec(d, block_idx_off=0):
    return pl.BlockSpec((BR, d), lambda i, o=block_idx_off: (i + o, 0))


def _full_spec(shape):
    return pl.BlockSpec(shape, lambda i: (0, 0))


def _dinv(degA, degB):
    indeg = degA[:, 0:1] + degB[:, 0:1]
    return lax.rsqrt(indeg + 1.0)


def _prescale_body(degA_ref, degB_ref, x_ref, xs_ref):
    dv = _dinv(degA_ref[...], degB_ref[...])
    xs_ref[...] = x_ref[...] * dv


def _step0_body(degA_ref, degB_ref, ypA_ref, ypB_ref, x_ref,
                wg0_ref, bg0_ref,
                w1_ref, b1_ref, w2p_ref, b2p_ref,
                c1_ref, cs1_ref):
    dv = _dinv(degA_ref[...], degB_ref[...])
    x = x_ref[...]
    yp = ypA_ref[...] + ypB_ref[...]
    y = dv * yp + (dv * dv) * x
    h = jnp.maximum(jnp.dot(y, wg0_ref[...], preferred_element_type=jnp.float32)
                    + bg0_ref[...], 0.0)
    z = jnp.maximum(jnp.dot(h, w1_ref[...], preferred_element_type=jnp.float32)
                    + b1_ref[...], 0.0)
    u16 = jnp.dot(z, w2p_ref[...], preferred_element_type=jnp.float32) + b2p_ref[...]
    coords0 = jnp.concatenate(
        [x[:, 126:128], jnp.zeros((x.shape[0], 14), jnp.float32)], axis=1)
    c1 = coords0 + u16
    c1_ref[...] = c1
    cs1_ref[...] = (dv * c1)[:, 0:2]


def _p12_body(degA_ref, degB_ref, ypA_ref, ypB_ref, x_ref,
              wgm1_ref, wgm2_ref, p1_ref, p2_ref):
    dv = _dinv(degA_ref[...], degB_ref[...])
    x = x_ref[...]
    y = dv * (ypA_ref[...] + ypB_ref[...]) + (dv * dv) * x
    p1_ref[...] = jnp.dot(y, wgm1_ref[...], preferred_element_type=jnp.float32)
    p2_ref[...] = jnp.dot(y, wgm2_ref[...], preferred_element_type=jnp.float32)


def _step_body(degA_ref, degB_ref, aA_ref, aB_ref, c_ref, p_ref,
               g2_ref, bg_ref, w1_ref, b1_ref, w2p_ref, b2p_ref,
               cn_ref, csn_ref):
    dv = _dinv(degA_ref[...], degB_ref[...])
    coords = c_ref[...]
    a = aA_ref[...] + aB_ref[...]
    ac = dv * a + (dv * dv) * coords
    h = jnp.maximum(jnp.dot(ac, g2_ref[...], preferred_element_type=jnp.float32)
                    + p_ref[...] + bg_ref[...], 0.0)
    z = jnp.maximum(jnp.dot(h, w1_ref[...], preferred_element_type=jnp.float32)
                    + b1_ref[...], 0.0)
    u16 = jnp.dot(z, w2p_ref[...], preferred_element_type=jnp.float32) + b2p_ref[...]
    cn = coords + u16
    cn_ref[...] = cn
    csn_ref[...] = (dv * cn)[:, 0:2]


def kernel(x, edge_index, Wg, bg, W1, b1, W2, b2, step_sizes):
    n, dfeat = x.shape
    h = Wg.shape[1]
    hh = W1.shape[1]
    n_pad = _pad_nodes(n)
    grid = (-(-n // BR),)

    ei = edge_index.astype(jnp.int32)
    src, dst = ei[0], ei[1]
    e = src.shape[0]

    nb = -(-e // (NW * K))
    e_pad = NW * nb * K
    src_p = jnp.concatenate([src, jnp.zeros((e_pad - e,), jnp.int32)])
    dst_p = jnp.concatenate([dst, jnp.full((e_pad - e,), n, jnp.int32)])
    src_t = src_p.reshape(NW, nb, K)
    dst_t = dst_p.reshape(NW, nb, K)
    srcf_t = src_p.reshape(NW, nb * K)

    k1 = 88
    e0 = int(e * 0.59)
    nb0 = -(-e0 // (NS * k1))
    nb1 = -(-(e - e0) // (NS * k1))

    def _part(arr, pad_val, eo, nbx):
        cap = NS * nbx * k1
        p = jnp.concatenate([arr, jnp.full((cap - arr.shape[0],), pad_val,
                                           jnp.int32)])
        return p.reshape(NS, nbx, k1)

    def _asym(arr, pad_val):
        p0 = _part(arr[:e0], pad_val, e0, nb0)
        p1 = _part(arr[e0:], pad_val, e - e0, nb1)
        p1 = jnp.concatenate(
            [p1, jnp.full((NS, nb0 - nb1, k1), pad_val, jnp.int32)], axis=1)
        return jnp.concatenate([p0, p1], axis=0)

    src_t1 = _asym(src, 0)
    dst_t1 = _asym(dst, n)

    rows_per_tile = n_pad // NS
    zeros128 = jnp.zeros((rows_per_tile, 128), jnp.float32)
    zeros16 = jnp.zeros((rows_per_tile, 16), jnp.float32)
    ones16 = jnp.ones((K, 16), jnp.float32)

    deg_seg = _make_sc_segsum(n_pad, 16, nb, gather=False)
    degp = deg_seg(ones16, dst_t, zeros16)

    nblk = n_pad // BR
    deg_specA = pl.BlockSpec((BR, 16), lambda i: (i, 0))
    deg_specB = pl.BlockSpec((BR, 16), lambda i, o=nblk: (i + o, 0))

    xs = pl.pallas_call(
        _prescale_body,
        grid=grid,
        in_specs=[deg_specA, deg_specB, _row_spec(dfeat)],
        out_specs=_row_spec(dfeat),
        out_shape=jax.ShapeDtypeStruct((n, dfeat), jnp.float32),
    )(degp, degp, x)

    seg128 = _make_sc_segsum(n_pad, dfeat, nb1, gather=True, k=k1, nb_hi=nb0)
    ypp = seg128(xs, src_t1, dst_t1, zeros128)

    mask126 = jnp.concatenate(
        [jnp.ones((126,), jnp.float32), jnp.zeros((2,), jnp.float32)])[:, None]
    wg0T = Wg[0].T
    wgm1T = Wg[1].T * mask126
    wgm2T = Wg[2].T * mask126
    g2 = [jnp.concatenate([Wg[s][:, 126:128].T,
                           jnp.zeros((14, h), jnp.float32)], axis=0)
          for s in range(NSTEPS)]
    w1T = [W1[s].T for s in range(NSTEPS)]
    w2pT = [jnp.concatenate([W2[s].T * step_sizes[s],
                             jnp.zeros((hh, 14), jnp.float32)], axis=1)
            for s in range(NSTEPS)]
    b2p = [jnp.concatenate([b2[s] * step_sizes[s],
                            jnp.zeros((14,), jnp.float32)])[None, :]
           for s in range(NSTEPS)]
    bg_r = [bg[s][None, :] for s in range(NSTEPS)]
    b1_r = [b1[s][None, :] for s in range(NSTEPS)]

    c1, cs1 = pl.pallas_call(
        _step0_body,
        grid=grid,
        in_specs=[deg_specA, deg_specB,
                  pl.BlockSpec((BR, dfeat), lambda i: (i, 0)),
                  pl.BlockSpec((BR, dfeat), lambda i, o=nblk: (i + o, 0)),
                  _row_spec(dfeat),
                  _full_spec((dfeat, h)), _full_spec((1, h)),
                  _full_spec((h, hh)), _full_spec((1, hh)),
                  _full_spec((hh, 16)), _full_spec((1, 16))],
        out_specs=[_row_spec(16), pl.BlockSpec((BR, 2), lambda i: (i, 0))],
        out_shape=[jax.ShapeDtypeStruct((n, 16), jnp.float32),
                   jax.ShapeDtypeStruct((n, 2), jnp.float32)],
    )(degp, degp, ypp, ypp, x,
      wg0T, bg_r[0], w1T[0], b1_r[0], w2pT[0], b2p[0])

    p1, p2 = pl.pallas_call(
        _p12_body,
        grid=grid,
        in_specs=[deg_specA, deg_specB,
                  pl.BlockSpec((BR, dfeat), lambda i: (i, 0)),
                  pl.BlockSpec((BR, dfeat), lambda i, o=nblk: (i + o, 0)),
                  _row_spec(dfeat),
                  _full_spec((dfeat, h)), _full_spec((dfeat, h))],
        out_specs=[_row_spec(h), _row_spec(h)],
        out_shape=[jax.ShapeDtypeStruct((n, h), jnp.float32),
                   jax.ShapeDtypeStruct((n, h), jnp.float32)],
    )(degp, degp, ypp, ypp, x, wgm1T, wgm2T)

    seg16 = _make_sc_segsum(n_pad, 16, nb, gather="local", n_rows=n)
    coords16, cs, ps = c1, cs1, [p1, p2]
    for s in (1, 2):
        ap = seg16(cs, srcf_t, dst_t, zeros16)
        coords16, cs = pl.pallas_call(
            _step_body,
            grid=grid,
            in_specs=[deg_specA, deg_specB,
                      pl.BlockSpec((BR, 16), lambda i: (i, 0)),
                      pl.BlockSpec((BR, 16), lambda i, o=nblk: (i + o, 0)),
                      _row_spec(16), _row_spec(h),
                      _full_spec((16, h)), _full_spec((1, h)),
                      _full_spec((h, hh)), _full_spec((1, hh)),
                      _full_spec((hh, 16)), _full_spec((1, 16))],
            out_specs=[_row_spec(16), pl.BlockSpec((BR, 2), lambda i: (i, 0))],
            out_shape=[jax.ShapeDtypeStruct((n, 16), jnp.float32),
                       jax.ShapeDtypeStruct((n, 2), jnp.float32)],
        )(degp, degp, ap, ap, coords16, ps[s - 1],
          g2[s], bg_r[s], w1T[s], b1_r[s], w2pT[s], b2p[s])

    return coords16[:, :2]


# speedup vs baseline: 1.1188x; 1.1188x over previous
import functools

import jax
import jax.numpy as jnp
from jax import lax
from jax.experimental import pallas as pl
from jax.experimental.pallas import tpu as pltpu
from jax.experimental.pallas import tpu_sc as plsc

NSTEPS = 3
NC, NS = 2, 16
NW = NC * NS
K = 96
BR = 2560


def _pad_nodes(n):
    import math
    lcm = math.lcm(NS * 8, BR)
    return -(-(n + 1) // lcm) * lcm


def _make_sc_segsum(n_pad, d, nb, gather, k=K, nb_hi=None, tc_tiling=False,
                    n_rows=None):
    rows_per_tile = n_pad // NS
    if nb_hi is None:
        nb_hi = nb
    mesh = plsc.VectorSubcoreMesh(core_axis_name="c", subcore_axis_name="s",
                                  num_cores=NC, num_subcores=NS)

    if gather is True:
        def body(vals_hbm, src_hbm, dst_hbm, zeros_hbm, out_hbm,
                 src_v, dst_v, rows_a, rows_b, acc, sg_a, sg_b, ss_a, ss_b):
            c = lax.axis_index("c")
            s = lax.axis_index("s")
            wid = c * NS + s
            pltpu.sync_copy(zeros_hbm, acc.at[pl.ds(s * rows_per_tile, rows_per_tile)])
            pltpu.sync_copy(src_hbm.at[wid], src_v)
            pltpu.sync_copy(dst_hbm.at[wid], dst_v)
            plsc.subcore_barrier()

            rows = (rows_a, rows_b)
            sg = (sg_a, sg_b)
            ss = (ss_a, ss_b)

            def pipeline(lo, hi):
                g_descs = [None] * hi
                s_descs = [None] * hi
                g_descs[lo] = pltpu.async_copy(
                    vals_hbm.at[src_v.at[lo]], rows[lo % 2], sg[lo % 2])
                for j in range(lo, hi):
                    b = j % 2
                    if j + 1 < hi:
                        if j - 1 >= lo:
                            s_descs[j - 1].wait()
                        g_descs[j + 1] = pltpu.async_copy(
                            vals_hbm.at[src_v.at[j + 1]], rows[1 - b], sg[1 - b])
                    g_descs[j].wait()
                    s_descs[j] = pltpu.async_copy(
                        rows[b], acc.at[dst_v.at[j]], ss[b], add=True)
                if hi - lo >= 2:
                    s_descs[hi - 2].wait()
                s_descs[hi - 1].wait()

            pipeline(0, nb)
            if nb_hi > nb:
                @pl.when(c == 0)
                def _():
                    pipeline(nb, nb_hi)

            plsc.subcore_barrier()
            off = c * n_pad + s * rows_per_tile
            pltpu.sync_copy(acc.at[pl.ds(s * rows_per_tile, rows_per_tile)],
                            out_hbm.at[pl.ds(off, rows_per_tile)])

        scratch = [
            pltpu.VMEM((nb_hi, k), jnp.int32),
            pltpu.VMEM((nb_hi, k), jnp.int32),
            pltpu.VMEM((k, d), jnp.float32),
            pltpu.VMEM((k, d), jnp.float32),
            pltpu.VMEM_SHARED((n_pad, d), jnp.float32),
            pltpu.SemaphoreType.DMA,
            pltpu.SemaphoreType.DMA,
            pltpu.SemaphoreType.DMA,
            pltpu.SemaphoreType.DMA,
        ]
    elif gather == "local":
        def body(cs_hbm, srcf_hbm, dst_hbm, zeros_hbm, out_hbm,
                 cs_v, srcf_v, dst_v, rows_a, rows_b, acc, ss_a, ss_b):
            c = lax.axis_index("c")
            s = lax.axis_index("s")
            wid = c * NS + s
            pltpu.sync_copy(zeros_hbm, acc.at[pl.ds(s * rows_per_tile, rows_per_tile)])
            pltpu.sync_copy(cs_hbm, cs_v)
            pltpu.sync_copy(srcf_hbm.at[wid], srcf_v)
            pltpu.sync_copy(dst_hbm.at[wid], dst_v)
            pltpu.sync_copy(zeros_hbm.at[pl.ds(0, k)], rows_a)
            pltpu.sync_copy(zeros_hbm.at[pl.ds(0, k)], rows_b)
            plsc.subcore_barrier()

            col0 = jnp.zeros((16,), jnp.int32)
            col1 = col0 + 1
            lane = lax.iota(jnp.int32, 16)

            def fill(rows_ref, j):
                for g in range(k // 16):
                    srcg = srcf_v[pl.ds(j * k + g * 16, 16)]
                    gx = plsc.load_gather(cs_v, [srcg, col0])
                    gy = plsc.load_gather(cs_v, [srcg, col1])
                    rowi = lane + (g * 16)
                    plsc.store_scatter(rows_ref, [rowi, col0], gx)
                    plsc.store_scatter(rows_ref, [rowi, col1], gy)

            rows = (rows_a, rows_b)
            ss = (ss_a, ss_b)
            s_descs = [None] * nb
            fill(rows[0], 0)
            for j in range(nb):
                b = j % 2
                s_descs[j] = pltpu.async_copy(
                    rows[b], acc.at[dst_v.at[j]], ss[b], add=True)
                if j + 1 < nb:
                    if j >= 1:
                        s_descs[j - 1].wait()
                    fill(rows[1 - b], j + 1)
            if nb >= 2:
                s_descs[nb - 2].wait()
            s_descs[nb - 1].wait()

            plsc.subcore_barrier()
            off = c * n_pad + s * rows_per_tile
            pltpu.sync_copy(acc.at[pl.ds(s * rows_per_tile, rows_per_tile)],
                            out_hbm.at[pl.ds(off, rows_per_tile)])

        scratch = [
            pltpu.VMEM((n_rows, 2), jnp.float32),
            pltpu.VMEM((nb * k,), jnp.int32),
            pltpu.VMEM((nb, k), jnp.int32),
            pltpu.VMEM((k, d), jnp.float32),
            pltpu.VMEM((k, d), jnp.float32),
            pltpu.VMEM_SHARED((n_pad, d), jnp.float32),
            pltpu.SemaphoreType.DMA,
            pltpu.SemaphoreType.DMA,
        ]
    else:
        def body(ones_hbm, dst_hbm, zeros_hbm, out_hbm,
                 dst_v, ones_v, acc):
            c = lax.axis_index("c")
            s = lax.axis_index("s")
            wid = c * NS + s
            pltpu.sync_copy(zeros_hbm, acc.at[pl.ds(s * rows_per_tile, rows_per_tile)])
            pltpu.sync_copy(dst_hbm.at[wid], dst_v)
            pltpu.sync_copy(ones_hbm, ones_v)
            plsc.subcore_barrier()

            def step(j, _):
                pltpu.sync_copy(ones_v, acc.at[dst_v.at[j]], add=True)
                return 0
            lax.fori_loop(0, nb, step, 0)

            plsc.subcore_barrier()
            off = c * n_pad + s * rows_per_tile
            pltpu.sync_copy(acc.at[pl.ds(s * rows_per_tile, rows_per_tile)],
                            out_hbm.at[pl.ds(off, rows_per_tile)])

        scratch = [
            pltpu.VMEM((nb, k), jnp.int32),
            pltpu.VMEM((k, d), jnp.float32),
            pltpu.VMEM_SHARED((n_pad, d), jnp.float32),
        ]

    return pl.kernel(
        body,
        out_type=jax.ShapeDtypeStruct((NC * n_pad, d), jnp.float32),
        mesh=mesh,
        scratch_types=scratch,
        compiler_params=pltpu.CompilerParams(
            use_tc_tiling_on_sc=tc_tiling,
            needs_layout_passes=(gather != "local")),
    )


def _row_spec(d, block_idx_off=0):
    return pl.BlockSpec((BR, d), lambda i, o=block_idx_off: (i + o, 0))


def _full_spec(shape):
    return pl.BlockSpec(shape, lambda i: (0, 0))


def _dinv(degA, degB):
    indeg = degA[:, 0:1] + degB[:, 0:1]
    return lax.rsqrt(indeg + 1.0)


def _prescale_body(degA_ref, degB_ref, x_ref, xs_ref):
    dv = _dinv(degA_ref[...], degB_ref[...])
    xs_ref[...] = x_ref[...] * dv


def _step0_body(degA_ref, degB_ref, ypA_ref, ypB_ref, x_ref,
                wg0_ref, bg0_ref,
                w1_ref, b1_ref, w2p_ref, b2p_ref,
                c1_ref, cs1_ref):
    dv = _dinv(degA_ref[...], degB_ref[...])
    x = x_ref[...]
    yp = ypA_ref[...] + ypB_ref[...]
    y = dv * yp + (dv * dv) * x
    h = jnp.maximum(jnp.dot(y, wg0_ref[...], preferred_element_type=jnp.float32)
                    + bg0_ref[...], 0.0)
    z = jnp.maximum(jnp.dot(h, w1_ref[...], preferred_element_type=jnp.float32)
                    + b1_ref[...], 0.0)
    u16 = jnp.dot(z, w2p_ref[...], preferred_element_type=jnp.float32) + b2p_ref[...]
    coords0 = jnp.concatenate(
        [x[:, 126:128], jnp.zeros((x.shape[0], 14), jnp.float32)], axis=1)
    c1 = coords0 + u16
    c1_ref[...] = c1
    cs1_ref[...] = (dv * c1)[:, 0:2]


def _p12_body(degA_ref, degB_ref, ypA_ref, ypB_ref, x_ref,
              wgm1_ref, wgm2_ref, p1_ref, p2_ref):
    dv = _dinv(degA_ref[...], degB_ref[...])
    x = x_ref[...]
    y = dv * (ypA_ref[...] + ypB_ref[...]) + (dv * dv) * x
    p1_ref[...] = jnp.dot(y, wgm1_ref[...], preferred_element_type=jnp.float32)
    p2_ref[...] = jnp.dot(y, wgm2_ref[...], preferred_element_type=jnp.float32)


def _step_body(degA_ref, degB_ref, aA_ref, aB_ref, c_ref, p_ref,
               g2_ref, bg_ref, w1_ref, b1_ref, w2p_ref, b2p_ref,
               cn_ref, csn_ref):
    dv = _dinv(degA_ref[...], degB_ref[...])
    coords = c_ref[...]
    a = aA_ref[...] + aB_ref[...]
    ac = dv * a + (dv * dv) * coords
    h = jnp.maximum(jnp.dot(ac, g2_ref[...], preferred_element_type=jnp.float32)
                    + p_ref[...] + bg_ref[...], 0.0)
    z = jnp.maximum(jnp.dot(h, w1_ref[...], preferred_element_type=jnp.float32)
                    + b1_ref[...], 0.0)
    u16 = jnp.dot(z, w2p_ref[...], preferred_element_type=jnp.float32) + b2p_ref[...]
    cn = coords + u16
    cn_ref[...] = cn
    csn_ref[...] = (dv * cn)[:, 0:2]


def kernel(x, edge_index, Wg, bg, W1, b1, W2, b2, step_sizes):
    n, dfeat = x.shape
    h = Wg.shape[1]
    hh = W1.shape[1]
    n_pad = _pad_nodes(n)
    grid = (-(-n // BR),)

    ei = edge_index.astype(jnp.int32)
    src, dst = ei[0], ei[1]
    e = src.shape[0]

    nb = -(-e // (NW * K))
    e_pad = NW * nb * K
    src_p = jnp.concatenate([src, jnp.zeros((e_pad - e,), jnp.int32)])
    dst_p = jnp.concatenate([dst, jnp.full((e_pad - e,), n, jnp.int32)])
    src_t = src_p.reshape(NW, nb, K)
    dst_t = dst_p.reshape(NW, nb, K)
    srcf_t = src_p.reshape(NW, nb * K)

    k1 = 88
    e0 = int(e * 0.59)
    nb0 = -(-e0 // (NS * k1))
    nb1 = -(-(e - e0) // (NS * k1))

    def _part(arr, pad_val, eo, nbx):
        cap = NS * nbx * k1
        p = jnp.concatenate([arr, jnp.full((cap - arr.shape[0],), pad_val,
                                           jnp.int32)])
        return p.reshape(NS, nbx, k1)

    def _asym(arr, pad_val):
        p0 = _part(arr[:e0], pad_val, e0, nb0)
        p1 = _part(arr[e0:], pad_val, e - e0, nb1)
        p1 = jnp.concatenate(
            [p1, jnp.full((NS, nb0 - nb1, k1), pad_val, jnp.int32)], axis=1)
        return jnp.concatenate([p0, p1], axis=0)

    src_t1 = _asym(src, 0)
    dst_t1 = _asym(dst, n)

    rows_per_tile = n_pad // NS
    zeros128 = jnp.zeros((rows_per_tile, 128), jnp.float32)
    zeros16 = jnp.zeros((rows_per_tile, 16), jnp.float32)
    ones16 = jnp.ones((K, 16), jnp.float32)

    deg_seg = _make_sc_segsum(n_pad, 16, nb, gather=False)
    degp = deg_seg(ones16, dst_t, zeros16)

    nblk = n_pad // BR
    deg_specA = pl.BlockSpec((BR, 16), lambda i: (i, 0))
    deg_specB = pl.BlockSpec((BR, 16), lambda i, o=nblk: (i + o, 0))

    xs = pl.pallas_call(
        _prescale_body,
        grid=grid,
        in_specs=[deg_specA, deg_specB, _row_spec(dfeat)],
        out_specs=_row_spec(dfeat),
        out_shape=jax.ShapeDtypeStruct((n, dfeat), jnp.float32),
    )(degp, degp, x)

    seg128 = _make_sc_segsum(n_pad, dfeat, nb1, gather=True, k=k1, nb_hi=nb0)
    ypp = seg128(xs, src_t1, dst_t1, zeros128)

    mask126 = jnp.concatenate(
        [jnp.ones((126,), jnp.float32), jnp.zeros((2,), jnp.float32)])[:, None]
    wg0T = Wg[0].T
    wgm1T = Wg[1].T * mask126
    wgm2T = Wg[2].T * mask126
    g2 = [jnp.concatenate([Wg[s][:, 126:128].T,
                           jnp.zeros((14, h), jnp.float32)], axis=0)
          for s in range(NSTEPS)]
    w1T = [W1[s].T for s in range(NSTEPS)]
    w2pT = [jnp.concatenate([W2[s].T * step_sizes[s],
                             jnp.zeros((hh, 14), jnp.float32)], axis=1)
            for s in range(NSTEPS)]
    b2p = [jnp.concatenate([b2[s] * step_sizes[s],
                            jnp.zeros((14,), jnp.float32)])[None, :]
           for s in range(NSTEPS)]
    bg_r = [bg[s][None, :] for s in range(NSTEPS)]
    b1_r = [b1[s][None, :] for s in range(NSTEPS)]

    c1, cs1 = pl.pallas_call(
        _step0_body,
        grid=grid,
        in_specs=[deg_specA, deg_specB,
                  pl.BlockSpec((BR, dfeat), lambda i: (i, 0)),
                  pl.BlockSpec((BR, dfeat), lambda i, o=nblk: (i + o, 0)),
                  _row_spec(dfeat),
                  _full_spec((dfeat, h)), _full_spec((1, h)),
                  _full_spec((h, hh)), _full_spec((1, hh)),
                  _full_spec((hh, 16)), _full_spec((1, 16))],
        out_specs=[_row_spec(16), pl.BlockSpec((BR, 2), lambda i: (i, 0))],
        out_shape=[jax.ShapeDtypeStruct((n, 16), jnp.float32),
                   jax.ShapeDtypeStruct((n, 2), jnp.float32)],
    )(degp, degp, ypp, ypp, x,
      wg0T, bg_r[0], w1T[0], b1_r[0], w2pT[0], b2p[0])

    p1, p2 = pl.pallas_call(
        _p12_body,
        grid=grid,
        in_specs=[deg_specA, deg_specB,
                  pl.BlockSpec((BR, dfeat), lambda i: (i, 0)),
                  pl.BlockSpec((BR, dfeat), lambda i, o=nblk: (i + o, 0)),
                  _row_spec(dfeat),
                  _full_spec((dfeat, h)), _full_spec((dfeat, h))],
        out_specs=[_row_spec(h), _row_spec(h)],
        out_shape=[jax.ShapeDtypeStruct((n, h), jnp.float32),
                   jax.ShapeDtypeStruct((n, h), jnp.float32)],
    )(degp, degp, ypp, ypp, x, wgm1T, wgm2T)

    seg16 = _make_sc_segsum(n_pad, 16, nb, gather="local", n_rows=n)
    coords16, cs, ps = c1, cs1, [p1, p2]
    for s in (1, 2):
        ap = seg16(cs, srcf_t, dst_t, zeros16)
        coords16, cs = pl.pallas_call(
            _step_body,
            grid=grid,
            in_specs=[deg_specA, deg_specB,
                      pl.BlockSpec((BR, 16), lambda i: (i, 0)),
                      pl.BlockSpec((BR, 16), lambda i, o=nblk: (i + o, 0)),
                      _row_spec(16), _row_spec(h),
                      _full_spec((16, h)), _full_spec((1, h)),
                      _full_spec((h, hh)), _full_spec((1, hh)),
                      _full_spec((hh, 16)), _full_spec((1, 16))],
            out_specs=[_row_spec(16), pl.BlockSpec((BR, 2), lambda i: (i, 0))],
            out_shape=[jax.ShapeDtypeStruct((n, 16), jnp.float32),
                       jax.ShapeDtypeStruct((n, 2), jnp.float32)],
        )(degp, degp, ap, ap, coords16, ps[s - 1],
          g2[s], bg_r[s], w1T[s], b1_r[s], w2pT[s], b2p[s])

    return coords16[:, :2]


# speedup vs baseline: 1.3282x; 1.1872x over previous
import functools

import jax
import jax.numpy as jnp
from jax import lax
from jax.experimental import pallas as pl
from jax.experimental.pallas import tpu as pltpu
from jax.experimental.pallas import tpu_sc as plsc

NSTEPS = 3
NC, NS = 2, 16
NW = NC * NS
K = 96
BR = 2560


def _pad_nodes(n):
    import math
    lcm = math.lcm(NS * 8, BR)
    return -(-(n + 1) // lcm) * lcm


def _make_sc_segsum(n_pad, d, nb, gather, k=K, nb_hi=None, tc_tiling=False,
                    n_rows=None):
    rows_per_tile = n_pad // NS
    if nb_hi is None:
        nb_hi = nb
    mesh = plsc.VectorSubcoreMesh(core_axis_name="c", subcore_axis_name="s",
                                  num_cores=NC, num_subcores=NS)

    if gather is True:
        def body(vals_hbm, src_hbm, dst_hbm, zeros_hbm, out_hbm,
                 src_v, dst_v, rows_a, rows_b, acc, sg_a, sg_b, ss_a, ss_b):
            c = lax.axis_index("c")
            s = lax.axis_index("s")
            wid = s * NC + c
            pltpu.sync_copy(zeros_hbm, acc.at[pl.ds(s * rows_per_tile, rows_per_tile)])
            pltpu.sync_copy(src_hbm.at[wid], src_v)
            pltpu.sync_copy(dst_hbm.at[wid], dst_v)
            plsc.subcore_barrier()

            rows = (rows_a, rows_b)
            sg = (sg_a, sg_b)
            ss = (ss_a, ss_b)

            def pipeline(lo, hi):
                g_descs = [None] * hi
                s_descs = [None] * hi
                g_descs[lo] = pltpu.async_copy(
                    vals_hbm.at[src_v.at[lo]], rows[lo % 2], sg[lo % 2])
                for j in range(lo, hi):
                    b = j % 2
                    if j + 1 < hi:
                        if j - 1 >= lo:
                            s_descs[j - 1].wait()
                        g_descs[j + 1] = pltpu.async_copy(
                            vals_hbm.at[src_v.at[j + 1]], rows[1 - b], sg[1 - b])
                    g_descs[j].wait()
                    s_descs[j] = pltpu.async_copy(
                        rows[b], acc.at[dst_v.at[j]], ss[b], add=True)
                if hi - lo >= 2:
                    s_descs[hi - 2].wait()
                s_descs[hi - 1].wait()

            pipeline(0, nb)
            if nb_hi > nb:
                @pl.when(c == 0)
                def _():
                    pipeline(nb, nb_hi)

            plsc.subcore_barrier()
            off = c * n_pad + s * rows_per_tile
            pltpu.sync_copy(acc.at[pl.ds(s * rows_per_tile, rows_per_tile)],
                            out_hbm.at[pl.ds(off, rows_per_tile)])

        scratch = [
            pltpu.VMEM((nb_hi, k), jnp.int32),
            pltpu.VMEM((nb_hi, k), jnp.int32),
            pltpu.VMEM((k, d), jnp.float32),
            pltpu.VMEM((k, d), jnp.float32),
            pltpu.VMEM_SHARED((n_pad, d), jnp.float32),
            pltpu.SemaphoreType.DMA,
            pltpu.SemaphoreType.DMA,
            pltpu.SemaphoreType.DMA,
            pltpu.SemaphoreType.DMA,
        ]
    elif gather == "local":
        def body(cs_hbm, srcf_hbm, dst_hbm, zeros_hbm, out_hbm,
                 cs_v, srcf_v, dst_v, rows_a, rows_b, acc, ss_a, ss_b):
            c = lax.axis_index("c")
            s = lax.axis_index("s")
            wid = s * NC + c
            pltpu.sync_copy(zeros_hbm, acc.at[pl.ds(s * rows_per_tile, rows_per_tile)])
            pltpu.sync_copy(cs_hbm, cs_v)
            pltpu.sync_copy(srcf_hbm.at[wid], srcf_v)
            pltpu.sync_copy(dst_hbm.at[wid], dst_v)
            pltpu.sync_copy(zeros_hbm.at[pl.ds(0, k)], rows_a)
            pltpu.sync_copy(zeros_hbm.at[pl.ds(0, k)], rows_b)
            plsc.subcore_barrier()

            col0 = jnp.zeros((16,), jnp.int32)
            col1 = col0 + 1
            lane = lax.iota(jnp.int32, 16)

            def fill(rows_ref, j):
                for g in range(k // 16):
                    srcg = srcf_v[pl.ds(j * k + g * 16, 16)]
                    gx = plsc.load_gather(cs_v, [srcg, col0])
                    gy = plsc.load_gather(cs_v, [srcg, col1])
                    rowi = lane + (g * 16)
                    plsc.store_scatter(rows_ref, [rowi, col0], gx)
                    plsc.store_scatter(rows_ref, [rowi, col1], gy)

            rows = (rows_a, rows_b)
            ss = (ss_a, ss_b)
            s_descs = [None] * nb
            fill(rows[0], 0)
            for j in range(nb):
                b = j % 2
                s_descs[j] = pltpu.async_copy(
                    rows[b], acc.at[dst_v.at[j]], ss[b], add=True)
                if j + 1 < nb:
                    if j >= 1:
                        s_descs[j - 1].wait()
                    fill(rows[1 - b], j + 1)
            if nb >= 2:
                s_descs[nb - 2].wait()
            s_descs[nb - 1].wait()

            plsc.subcore_barrier()
            off = c * n_pad + s * rows_per_tile
            pltpu.sync_copy(acc.at[pl.ds(s * rows_per_tile, rows_per_tile)],
                            out_hbm.at[pl.ds(off, rows_per_tile)])

        scratch = [
            pltpu.VMEM((n_rows, 2), jnp.float32),
            pltpu.VMEM((nb * k,), jnp.int32),
            pltpu.VMEM((nb, k), jnp.int32),
            pltpu.VMEM((k, d), jnp.float32),
            pltpu.VMEM((k, d), jnp.float32),
            pltpu.VMEM_SHARED((n_pad, d), jnp.float32),
            pltpu.SemaphoreType.DMA,
            pltpu.SemaphoreType.DMA,
        ]
    else:
        def body(ones_hbm, dst_hbm, zeros_hbm, out_hbm,
                 dst_v, ones_v, acc):
            c = lax.axis_index("c")
            s = lax.axis_index("s")
            wid = s * NC + c
            pltpu.sync_copy(zeros_hbm, acc.at[pl.ds(s * rows_per_tile, rows_per_tile)])
            pltpu.sync_copy(dst_hbm.at[wid], dst_v)
            pltpu.sync_copy(ones_hbm, ones_v)
            plsc.subcore_barrier()

            def step(j, _):
                pltpu.sync_copy(ones_v, acc.at[dst_v.at[j]], add=True)
                return 0
            lax.fori_loop(0, nb, step, 0)

            plsc.subcore_barrier()
            off = c * n_pad + s * rows_per_tile
            pltpu.sync_copy(acc.at[pl.ds(s * rows_per_tile, rows_per_tile)],
                            out_hbm.at[pl.ds(off, rows_per_tile)])

        scratch = [
            pltpu.VMEM((nb, k), jnp.int32),
            pltpu.VMEM((k, d), jnp.float32),
            pltpu.VMEM_SHARED((n_pad, d), jnp.float32),
        ]

    return pl.kernel(
        body,
        out_type=jax.ShapeDtypeStruct((NC * n_pad, d), jnp.float32),
        mesh=mesh,
        scratch_types=scratch,
        compiler_params=pltpu.CompilerParams(
            use_tc_tiling_on_sc=tc_tiling,
            needs_layout_passes=(gather != "local")),
    )


def _row_spec(d, block_idx_off=0):
    return pl.BlockSpec((BR, d), lambda i, o=block_idx_off: (i + o, 0))


def _full_spec(shape):
    return pl.BlockSpec(shape, lambda i: (0, 0))


def _dinv(degA, degB):
    indeg = degA[:, 0:1] + degB[:, 0:1]
    return lax.rsqrt(indeg + 1.0)


def _prescale_body(degA_ref, degB_ref, x_ref, xs_ref):
    dv = _dinv(degA_ref[...], degB_ref[...])
    xs_ref[...] = x_ref[...] * dv


def _step0_body(degA_ref, degB_ref, ypA_ref, ypB_ref, x_ref,
                wg0_ref, bg0_ref,
                w1_ref, b1_ref, w2p_ref, b2p_ref,
                c1_ref, cs1_ref):
    dv = _dinv(degA_ref[...], degB_ref[...])
    x = x_ref[...]
    yp = ypA_ref[...] + ypB_ref[...]
    y = dv * yp + (dv * dv) * x
    h = jnp.maximum(jnp.dot(y, wg0_ref[...], preferred_element_type=jnp.float32)
                    + bg0_ref[...], 0.0)
    z = jnp.maximum(jnp.dot(h, w1_ref[...], preferred_element_type=jnp.float32)
                    + b1_ref[...], 0.0)
    u16 = jnp.dot(z, w2p_ref[...], preferred_element_type=jnp.float32) + b2p_ref[...]
    coords0 = jnp.concatenate(
        [x[:, 126:128], jnp.zeros((x.shape[0], 14), jnp.float32)], axis=1)
    c1 = coords0 + u16
    c1_ref[...] = c1
    cs1_ref[...] = (dv * c1)[:, 0:2]


def _p12_body(degA_ref, degB_ref, ypA_ref, ypB_ref, x_ref,
              wgm1_ref, wgm2_ref, p1_ref, p2_ref):
    dv = _dinv(degA_ref[...], degB_ref[...])
    x = x_ref[...]
    y = dv * (ypA_ref[...] + ypB_ref[...]) + (dv * dv) * x
    p1_ref[...] = jnp.dot(y, wgm1_ref[...], preferred_element_type=jnp.float32)
    p2_ref[...] = jnp.dot(y, wgm2_ref[...], preferred_element_type=jnp.float32)


def _step_body(degA_ref, degB_ref, aA_ref, aB_ref, c_ref, p_ref,
               g2_ref, bg_ref, w1_ref, b1_ref, w2p_ref, b2p_ref,
               cn_ref, csn_ref):
    dv = _dinv(degA_ref[...], degB_ref[...])
    coords = c_ref[...]
    a = aA_ref[...] + aB_ref[...]
    ac = dv * a + (dv * dv) * coords
    h = jnp.maximum(jnp.dot(ac, g2_ref[...], preferred_element_type=jnp.float32)
                    + p_ref[...] + bg_ref[...], 0.0)
    z = jnp.maximum(jnp.dot(h, w1_ref[...], preferred_element_type=jnp.float32)
                    + b1_ref[...], 0.0)
    u16 = jnp.dot(z, w2p_ref[...], preferred_element_type=jnp.float32) + b2p_ref[...]
    cn = coords + u16
    cn_ref[...] = cn
    csn_ref[...] = (dv * cn)[:, 0:2]


def kernel(x, edge_index, Wg, bg, W1, b1, W2, b2, step_sizes):
    n, dfeat = x.shape
    h = Wg.shape[1]
    hh = W1.shape[1]
    n_pad = _pad_nodes(n)
    grid = (-(-n // BR),)

    ei = edge_index.astype(jnp.int32)
    src, dst = ei[0], ei[1]
    e = src.shape[0]

    nb = -(-e // (NW * K))
    e_pad = NW * nb * K
    src_p = jnp.concatenate([src, jnp.zeros((e_pad - e,), jnp.int32)])
    dst_p = jnp.concatenate([dst, jnp.full((e_pad - e,), n, jnp.int32)])
    src_t = src_p.reshape(NW, nb, K)
    dst_t = dst_p.reshape(NW, nb, K)
    srcf_t = src_p.reshape(NW, nb * K)

    k1 = 88
    e0 = int(e * 0.62)
    nb0 = -(-e0 // (NS * k1))
    nb1 = -(-(e - e0) // (NS * k1))

    def _part(arr, pad_val, eo, nbx):
        cap = NS * nbx * k1
        p = jnp.concatenate([arr, jnp.full((cap - arr.shape[0],), pad_val,
                                           jnp.int32)])
        return p.reshape(NS, nbx, k1)

    def _asym(arr, pad_val):
        p0 = _part(arr[:e0], pad_val, e0, nb0)
        p1 = _part(arr[e0:], pad_val, e - e0, nb1)
        p1 = jnp.concatenate(
            [p1, jnp.full((NS, nb0 - nb1, k1), pad_val, jnp.int32)], axis=1)
        return jnp.stack([p0, p1], axis=1).reshape(NW, nb0, k1)

    src_t1 = _asym(src, 0)
    dst_t1 = _asym(dst, n)

    rows_per_tile = n_pad // NS
    zeros128 = jnp.zeros((rows_per_tile, 128), jnp.float32)
    zeros16 = jnp.zeros((rows_per_tile, 16), jnp.float32)
    ones16 = jnp.ones((K, 16), jnp.float32)

    deg_seg = _make_sc_segsum(n_pad, 16, nb, gather=False)
    degp = deg_seg(ones16, dst_t, zeros16)

    nblk = n_pad // BR
    deg_specA = pl.BlockSpec((BR, 16), lambda i: (i, 0))
    deg_specB = pl.BlockSpec((BR, 16), lambda i, o=nblk: (i + o, 0))

    xs = pl.pallas_call(
        _prescale_body,
        grid=grid,
        in_specs=[deg_specA, deg_specB, _row_spec(dfeat)],
        out_specs=_row_spec(dfeat),
        out_shape=jax.ShapeDtypeStruct((n, dfeat), jnp.float32),
    )(degp, degp, x)

    seg128 = _make_sc_segsum(n_pad, dfeat, nb1, gather=True, k=k1, nb_hi=nb0)
    ypp = seg128(xs, src_t1, dst_t1, zeros128)

    mask126 = jnp.concatenate(
        [jnp.ones((126,), jnp.float32), jnp.zeros((2,), jnp.float32)])[:, None]
    wg0T = Wg[0].T
    wgm1T = Wg[1].T * mask126
    wgm2T = Wg[2].T * mask126
    g2 = [jnp.concatenate([Wg[s][:, 126:128].T,
                           jnp.zeros((14, h), jnp.float32)], axis=0)
          for s in range(NSTEPS)]
    w1T = [W1[s].T for s in range(NSTEPS)]
    w2pT = [jnp.concatenate([W2[s].T * step_sizes[s],
                             jnp.zeros((hh, 14), jnp.float32)], axis=1)
            for s in range(NSTEPS)]
    b2p = [jnp.concatenate([b2[s] * step_sizes[s],
                            jnp.zeros((14,), jnp.float32)])[None, :]
           for s in range(NSTEPS)]
    bg_r = [bg[s][None, :] for s in range(NSTEPS)]
    b1_r = [b1[s][None, :] for s in range(NSTEPS)]

    c1, cs1 = pl.pallas_call(
        _step0_body,
        grid=grid,
        in_specs=[deg_specA, deg_specB,
                  pl.BlockSpec((BR, dfeat), lambda i: (i, 0)),
                  pl.BlockSpec((BR, dfeat), lambda i, o=nblk: (i + o, 0)),
                  _row_spec(dfeat),
                  _full_spec((dfeat, h)), _full_spec((1, h)),
                  _full_spec((h, hh)), _full_spec((1, hh)),
                  _full_spec((hh, 16)), _full_spec((1, 16))],
        out_specs=[_row_spec(16), pl.BlockSpec((BR, 2), lambda i: (i, 0))],
        out_shape=[jax.ShapeDtypeStruct((n, 16), jnp.float32),
                   jax.ShapeDtypeStruct((n, 2), jnp.float32)],
    )(degp, degp, ypp, ypp, x,
      wg0T, bg_r[0], w1T[0], b1_r[0], w2pT[0], b2p[0])

    p1, p2 = pl.pallas_call(
        _p12_body,
        grid=grid,
        in_specs=[deg_specA, deg_specB,
                  pl.BlockSpec((BR, dfeat), lambda i: (i, 0)),
                  pl.BlockSpec((BR, dfeat), lambda i, o=nblk: (i + o, 0)),
                  _row_spec(dfeat),
                  _full_spec((dfeat, h)), _full_spec((dfeat, h))],
        out_specs=[_row_spec(h), _row_spec(h)],
        out_shape=[jax.ShapeDtypeStruct((n, h), jnp.float32),
                   jax.ShapeDtypeStruct((n, h), jnp.float32)],
    )(degp, degp, ypp, ypp, x, wgm1T, wgm2T)

    seg16 = _make_sc_segsum(n_pad, 16, nb, gather="local", n_rows=n)
    coords16, cs, ps = c1, cs1, [p1, p2]
    for s in (1, 2):
        ap = seg16(cs, srcf_t, dst_t, zeros16)
        coords16, cs = pl.pallas_call(
            _step_body,
            grid=grid,
            in_specs=[deg_specA, deg_specB,
                      pl.BlockSpec((BR, 16), lambda i: (i, 0)),
                      pl.BlockSpec((BR, 16), lambda i, o=nblk: (i + o, 0)),
                      _row_spec(16), _row_spec(h),
                      _full_spec((16, h)), _full_spec((1, h)),
                      _full_spec((h, hh)), _full_spec((1, hh)),
                      _full_spec((hh, 16)), _full_spec((1, 16))],
            out_specs=[_row_spec(16), pl.BlockSpec((BR, 2), lambda i: (i, 0))],
            out_shape=[jax.ShapeDtypeStruct((n, 16), jnp.float32),
                       jax.ShapeDtypeStruct((n, 2), jnp.float32)],
        )(degp, degp, ap, ap, coords16, ps[s - 1],
          g2[s], bg_r[s], w1T[s], b1_r[s], w2pT[s], b2p[s])

    return coords16[:, :2]
